# Initial kernel scaffold; baseline (speedup 1.0000x reference)
import jax
import jax.numpy as jnp
from jax.experimental import pallas as pl


def kernel(x):
    raise NotImplementedError("write your pallas kernel here")


# speedup vs baseline: 32.7245x; 32.7245x over previous
import jax
import jax.numpy as jnp
from jax import lax
from jax.experimental import pallas as pl
from jax.experimental.pallas import tpu as pltpu

_R = 64 * 96
_C0, _C1 = 128, 300
_W = _C1 - _C0
_N = _R * _W
_K = int(0.1 * _N)
_MSB = 0x80000000


def _monotone_u32(xs):
    u = lax.bitcast_convert_type(xs, jnp.uint32)
    msb = jnp.uint32(_MSB)
    return jnp.where(u >= msb, ~u, u ^ msb)


def _body(x_ref, o_ref, mu_ref):
    xs = x_ref[:, _C0:_C1]
    mu_ref[...] = _monotone_u32(xs)
    xmax = jnp.max(xs)

    def bit_step(i, ans):
        shift = (31 - i).astype(jnp.uint32)
        cand = ans | lax.shift_left(jnp.uint32(1), shift)
        cnt = jnp.sum((mu_ref[...] < cand).astype(jnp.int32))
        return jnp.where(cnt <= _K, cand, ans)

    ans = lax.fori_loop(0, 32, bit_step, jnp.uint32(0))
    msb = jnp.uint32(_MSB)
    b_eps = jnp.where(ans >= msb, ans ^ msb, ~ans)
    eps = lax.bitcast_convert_type(b_eps, jnp.float32)

    t = jnp.maximum(eps, jnp.float32(0.001))
    lo = jnp.log10(t)
    hi = jnp.log10(jnp.maximum(xmax, t))
    inv = 1.0 / (hi - lo)
    y = jnp.log10(jnp.maximum(xs, t))
    o_ref[...] = (y - lo) * inv


def kernel(x):
    x2 = x.reshape(_R, 512)
    out = pl.pallas_call(
        _body,
        out_shape=jax.ShapeDtypeStruct((_R, _W), jnp.float32),
        scratch_shapes=[pltpu.VMEM((_R, _W), jnp.uint32)],
    )(x2)
    return out.reshape(x.shape[0], x.shape[1], _W)
